# baseline (speedup 1.0000x reference)
import jax
import jax.numpy as jnp
import numpy as np
from jax.experimental import pallas as pl
from jax.experimental.pallas import tpu as pltpu

B = 4
HP = 28
WP = 28
D = 384
M = 16384
K_NN = 9
OUT = 224
P = B * HP * WP
P_PAD = 3328
PBLK = 256
CHUNK = 2048
N_CHUNK = M // CHUNK
NP_B = HP * WP
N_BLK = P_PAD // PBLK


def _build_L():
    out, inp = OUT, HP
    scale = inp / out
    Rm = np.zeros((out, inp), np.float64)
    for i in range(out):
        c = (i + 0.5) * scale - 0.5
        f = int(np.floor(c))
        w = c - f
        f0 = min(max(f, 0), inp - 1)
        f1 = min(max(f + 1, 0), inp - 1)
        Rm[i, f0] += 1 - w
        Rm[i, f1] += w
    sigma = 4.0
    radius = int(4.0 * sigma + 0.5)
    xs = np.arange(-radius, radius + 1).astype(np.float32)
    k = np.exp(-0.5 * (xs / sigma) ** 2)
    k = k / k.sum()
    G = np.zeros((out, out), np.float64)
    for i in range(out):
        for t in range(-radius, radius + 1):
            j = i + t
            if 0 <= j < out:
                G[i, j] += k[t + radius]
    return (G @ Rm).astype(np.float32)


_L = _build_L()


def _k1_body(emb_ref, mb_ref, ps_ref, xnr_ref, yn_ref, mbs_ref):
    i = pl.program_id(0)

    emb = emb_ref[...]
    xb = emb.astype(jnp.bfloat16)
    xn = jnp.sum(emb * emb, axis=1).reshape(PBLK, 1)
    xnr_ref[...] = xn.reshape(1, 1, PBLK)

    acc = jnp.full((PBLK, 1), jnp.inf, jnp.float32)
    for c in range(N_CHUNK):
        sl = pl.ds(c * CHUNK, CHUNK)

        @pl.when(i == 0)
        def _(sl=sl):
            mb_c_f = mb_ref[sl, :]
            yn_ref[0, sl] = jnp.sum(mb_c_f * mb_c_f, axis=1)
            mbs_ref[sl, :] = (-2.0 * mb_c_f).astype(jnp.bfloat16)

        mb_c = mbs_ref[sl, :]
        mm = jax.lax.dot_general(
            xb, mb_c, (((1,), (1,)), ((), ())),
            preferred_element_type=jnp.float32)
        yn_c = yn_ref[0, sl].reshape(1, CHUNK)
        part = jnp.min(mm + yn_c, axis=1, keepdims=True)
        acc = jnp.minimum(acc, part)
    ps = jnp.sqrt(jnp.maximum(acc + xn, 0.0))
    ps_ref[...] = ps.reshape(1, 1, PBLK)


def _k1(emb_pad, mb):
    ps, xnr, yn, mbs = pl.pallas_call(
        _k1_body,
        grid=(N_BLK,),
        in_specs=[
            pl.BlockSpec((PBLK, D), lambda i: (i, 0)),
            pl.BlockSpec((M, D), lambda i: (0, 0)),
        ],
        out_specs=(
            pl.BlockSpec((1, 1, PBLK), lambda i: (i, 0, 0)),
            pl.BlockSpec((1, 1, PBLK), lambda i: (i, 0, 0)),
            pl.BlockSpec((1, M), lambda i: (0, 0)),
            pl.BlockSpec((M, D), lambda i: (0, 0)),
        ),
        out_shape=(
            jax.ShapeDtypeStruct((N_BLK, 1, PBLK), jnp.float32),
            jax.ShapeDtypeStruct((N_BLK, 1, PBLK), jnp.float32),
            jax.ShapeDtypeStruct((1, M), jnp.float32),
            jax.ShapeDtypeStruct((M, D), jnp.bfloat16),
        ),
    )(emb_pad, mb)
    return ps, xnr, yn, mbs


def _argmin_row(vals, iota):
    m = jnp.min(vals, axis=1, keepdims=True)
    idx = jnp.min(jnp.where(vals == m, iota, jnp.int32(2 ** 30)),
                  axis=1, keepdims=True)
    return m, idx, iota == idx


def _k2_body(ps4_ref, psimg_ref, emb_ref, mbs_ref, xn_row_ref, yn_ref,
             l_ref, lt_ref, amap_ref, pred_ref):
    ps4 = ps4_ref[...]
    iota_p = jax.lax.broadcasted_iota(jnp.int32, (B, NP_B), 1)
    smax, pidx, oh_p = _argmin_row(-ps4, iota_p)
    score = -smax

    xb = emb_ref[...].astype(jnp.bfloat16)
    iota_gp = jax.lax.broadcasted_iota(jnp.int32, (B, P_PAD), 1)
    goff = jax.lax.broadcasted_iota(jnp.int32, (B, 1), 0) * NP_B + pidx
    oh_gp = (iota_gp == goff)
    max_feat = jax.lax.dot_general(
        oh_gp.astype(jnp.bfloat16), xb, (((1,), (0,)), ((), ())),
        preferred_element_type=jnp.float32).astype(jnp.bfloat16)
    xn_sel = jnp.max(jnp.where(oh_gp, xn_row_ref[...], -jnp.inf),
                     axis=1, keepdims=True)

    mbs = mbs_ref[...]
    yn = yn_ref[...]

    mm1 = jax.lax.dot_general(
        max_feat, mbs, (((1,), (1,)), ((), ())),
        preferred_element_type=jnp.float32)
    res1 = (xn_sel + mm1) + yn
    drow = jnp.sqrt(jnp.maximum(res1, 0.0))

    iota_m = jax.lax.broadcasted_iota(jnp.int32, (B, M), 1)
    _, _, oh_nn = _argmin_row(res1, iota_m)

    nn_feat = (-0.5 * jax.lax.dot_general(
        oh_nn.astype(jnp.bfloat16), mbs, (((1,), (0,)), ((), ())),
        preferred_element_type=jnp.float32)).astype(jnp.bfloat16)
    nn_norm = jnp.max(jnp.where(oh_nn, yn, -jnp.inf), axis=1, keepdims=True)

    mm2 = jax.lax.dot_general(
        nn_feat, mbs, (((1,), (1,)), ((), ())),
        preferred_element_type=jnp.float32)
    d2 = jnp.sqrt(jnp.maximum((nn_norm + mm2) + yn, 0.0))

    cols = []
    d2m = d2
    for _ in range(K_NN):
        _, _, oh_k = _argmin_row(d2m, iota_m)
        dk = jnp.max(jnp.where(oh_k, drow, -jnp.inf), axis=1, keepdims=True)
        cols.append(dk)
        d2m = jnp.where(oh_k, jnp.inf, d2m)
    dists9 = jnp.concatenate(cols, axis=1)

    mx = jnp.max(dists9, axis=1, keepdims=True)
    e = jnp.exp(dists9 - mx)
    p0 = e[:, 0:1] / jnp.sum(e, axis=1, keepdims=True)
    pred_ref[...] = (1.0 - p0) * score

    l_mat = l_ref[...]
    lt_mat = lt_ref[...]
    for b in range(B):
        blk = psimg_ref[pl.ds(b * HP, HP), :]
        t1 = jax.lax.dot_general(
            l_mat, blk, (((1,), (0,)), ((), ())),
            preferred_element_type=jnp.float32,
            precision=jax.lax.Precision.HIGHEST)
        t2 = jax.lax.dot_general(
            t1, lt_mat, (((1,), (0,)), ((), ())),
            preferred_element_type=jnp.float32,
            precision=jax.lax.Precision.HIGHEST)
        amap_ref[pl.ds(b * OUT, OUT), :] = t2


def _k2(ps4, psimg, emb_pad, mbs, xn_row, yn, l_mat, lt_mat):
    return pl.pallas_call(
        _k2_body,
        out_shape=(
            jax.ShapeDtypeStruct((B * OUT, OUT), jnp.float32),
            jax.ShapeDtypeStruct((B, 1), jnp.float32),
        ),
    )(ps4, psimg, emb_pad, mbs, xn_row, yn, l_mat, lt_mat)


@jax.jit
def kernel(embedding, memory_bank):
    emb_pad = jnp.pad(embedding, ((0, P_PAD - P), (0, 0)))
    ps3, xnr3, yn, mbs = _k1(emb_pad, memory_bank)
    ps = ps3.reshape(P_PAD)[:P]
    xn_row = xnr3.reshape(1, P_PAD)

    ps4 = ps.reshape(B, NP_B)
    psimg = ps.reshape(B * HP, WP)
    l_mat = jnp.asarray(_L)
    lt_mat = l_mat.T
    amap, pred = _k2(ps4, psimg, emb_pad, mbs, xn_row, yn, l_mat, lt_mat)
    return amap.reshape(B, 1, OUT, OUT), pred.reshape(B)


# speedup vs baseline: 1.3631x; 1.3631x over previous
import jax
import jax.numpy as jnp
import numpy as np
from jax.experimental import pallas as pl
from jax.experimental.pallas import tpu as pltpu

B = 4
HP = 28
WP = 28
D = 384
M = 16384
K_NN = 9
OUT = 224
P = B * HP * WP
P_PAD = 3328
PBLK = 256
CHUNK = 4096
N_CHUNK = M // CHUNK
NP_B = HP * WP
N_BLK = P_PAD // PBLK


def _build_L():
    out, inp = OUT, HP
    scale = inp / out
    Rm = np.zeros((out, inp), np.float64)
    for i in range(out):
        c = (i + 0.5) * scale - 0.5
        f = int(np.floor(c))
        w = c - f
        f0 = min(max(f, 0), inp - 1)
        f1 = min(max(f + 1, 0), inp - 1)
        Rm[i, f0] += 1 - w
        Rm[i, f1] += w
    sigma = 4.0
    radius = int(4.0 * sigma + 0.5)
    xs = np.arange(-radius, radius + 1).astype(np.float32)
    k = np.exp(-0.5 * (xs / sigma) ** 2)
    k = k / k.sum()
    G = np.zeros((out, out), np.float64)
    for i in range(out):
        for t in range(-radius, radius + 1):
            j = i + t
            if 0 <= j < out:
                G[i, j] += k[t + radius]
    return (G @ Rm).astype(np.float32)


_L = _build_L()


def _k1_body(emb_ref, mb_ref, ps_ref, xnr_ref, yn_ref, mbs_ref):
    i = pl.program_id(0)

    @pl.when(i == 0)
    def _():
        mb = mb_ref[...]
        yn_ref[...] = jnp.sum(mb * mb, axis=1).reshape(1, M)
        mbs_ref[...] = (-2.0 * mb).astype(jnp.bfloat16)

    emb = emb_ref[...]
    xb = emb.astype(jnp.bfloat16)
    xn = jnp.sum(emb * emb, axis=1).reshape(PBLK, 1)
    xnr_ref[...] = xn.reshape(1, 1, PBLK)

    acc = jnp.full((PBLK, 1), jnp.inf, jnp.float32)
    for c in range(N_CHUNK):
        mb_c = mbs_ref[pl.ds(c * CHUNK, CHUNK), :]
        mm = jax.lax.dot_general(
            xb, mb_c, (((1,), (1,)), ((), ())),
            preferred_element_type=jnp.float32)
        yn_c = yn_ref[0, pl.ds(c * CHUNK, CHUNK)].reshape(1, CHUNK)
        part = jnp.min(mm + yn_c, axis=1, keepdims=True)
        acc = jnp.minimum(acc, part)
    ps = jnp.sqrt(jnp.maximum(acc + xn, 0.0))
    ps_ref[...] = ps.reshape(1, 1, PBLK)


def _k1(emb_pad, mb):
    ps, xnr, yn, mbs = pl.pallas_call(
        _k1_body,
        grid=(N_BLK,),
        in_specs=[
            pl.BlockSpec((PBLK, D), lambda i: (i, 0)),
            pl.BlockSpec((M, D), lambda i: (0, 0)),
        ],
        out_specs=(
            pl.BlockSpec((1, 1, PBLK), lambda i: (i, 0, 0)),
            pl.BlockSpec((1, 1, PBLK), lambda i: (i, 0, 0)),
            pl.BlockSpec((1, M), lambda i: (0, 0)),
            pl.BlockSpec((M, D), lambda i: (0, 0)),
        ),
        out_shape=(
            jax.ShapeDtypeStruct((N_BLK, 1, PBLK), jnp.float32),
            jax.ShapeDtypeStruct((N_BLK, 1, PBLK), jnp.float32),
            jax.ShapeDtypeStruct((1, M), jnp.float32),
            jax.ShapeDtypeStruct((M, D), jnp.bfloat16),
        ),
    )(emb_pad, mb)
    return ps, xnr, yn, mbs


def _argmin_row(vals, iota):
    m = jnp.min(vals, axis=1, keepdims=True)
    idx = jnp.min(jnp.where(vals == m, iota, jnp.int32(2 ** 30)),
                  axis=1, keepdims=True)
    return m, idx, iota == idx


def _k2_body(ps4_ref, psimg_ref, emb_ref, mbs_ref, xn_row_ref, yn_ref,
             l_ref, lt_ref, amap_ref, pred_ref):
    ps4 = ps4_ref[...]
    iota_p = jax.lax.broadcasted_iota(jnp.int32, (B, NP_B), 1)
    smax, pidx, oh_p = _argmin_row(-ps4, iota_p)
    score = -smax

    xb = emb_ref[...].astype(jnp.bfloat16)
    iota_gp = jax.lax.broadcasted_iota(jnp.int32, (B, P_PAD), 1)
    goff = jax.lax.broadcasted_iota(jnp.int32, (B, 1), 0) * NP_B + pidx
    oh_gp = (iota_gp == goff)
    max_feat = jax.lax.dot_general(
        oh_gp.astype(jnp.bfloat16), xb, (((1,), (0,)), ((), ())),
        preferred_element_type=jnp.float32).astype(jnp.bfloat16)
    xn_sel = jnp.max(jnp.where(oh_gp, xn_row_ref[...], -jnp.inf),
                     axis=1, keepdims=True)

    mbs = mbs_ref[...]
    yn = yn_ref[...]

    mm1 = jax.lax.dot_general(
        max_feat, mbs, (((1,), (1,)), ((), ())),
        preferred_element_type=jnp.float32)
    res1 = (xn_sel + mm1) + yn
    drow = jnp.sqrt(jnp.maximum(res1, 0.0))

    iota_m = jax.lax.broadcasted_iota(jnp.int32, (B, M), 1)
    _, _, oh_nn = _argmin_row(res1, iota_m)

    nn_feat = (-0.5 * jax.lax.dot_general(
        oh_nn.astype(jnp.bfloat16), mbs, (((1,), (0,)), ((), ())),
        preferred_element_type=jnp.float32)).astype(jnp.bfloat16)
    nn_norm = jnp.max(jnp.where(oh_nn, yn, -jnp.inf), axis=1, keepdims=True)

    mm2 = jax.lax.dot_general(
        nn_feat, mbs, (((1,), (1,)), ((), ())),
        preferred_element_type=jnp.float32)
    d2 = jnp.sqrt(jnp.maximum((nn_norm + mm2) + yn, 0.0))

    cols = []
    d2m = d2
    for _ in range(K_NN):
        _, _, oh_k = _argmin_row(d2m, iota_m)
        dk = jnp.max(jnp.where(oh_k, drow, -jnp.inf), axis=1, keepdims=True)
        cols.append(dk)
        d2m = jnp.where(oh_k, jnp.inf, d2m)
    dists9 = jnp.concatenate(cols, axis=1)

    mx = jnp.max(dists9, axis=1, keepdims=True)
    e = jnp.exp(dists9 - mx)
    p0 = e[:, 0:1] / jnp.sum(e, axis=1, keepdims=True)
    pred_ref[...] = (1.0 - p0) * score

    l_mat = l_ref[...]
    lt_mat = lt_ref[...]
    for b in range(B):
        blk = psimg_ref[pl.ds(b * HP, HP), :]
        t1 = jax.lax.dot_general(
            l_mat, blk, (((1,), (0,)), ((), ())),
            preferred_element_type=jnp.float32,
            precision=jax.lax.Precision.HIGHEST)
        t2 = jax.lax.dot_general(
            t1, lt_mat, (((1,), (0,)), ((), ())),
            preferred_element_type=jnp.float32,
            precision=jax.lax.Precision.HIGHEST)
        amap_ref[pl.ds(b * OUT, OUT), :] = t2


def _k2(ps4, psimg, emb_pad, mbs, xn_row, yn, l_mat, lt_mat):
    return pl.pallas_call(
        _k2_body,
        out_shape=(
            jax.ShapeDtypeStruct((B * OUT, OUT), jnp.float32),
            jax.ShapeDtypeStruct((B, 1), jnp.float32),
        ),
    )(ps4, psimg, emb_pad, mbs, xn_row, yn, l_mat, lt_mat)


@jax.jit
def kernel(embedding, memory_bank):
    emb_pad = jnp.pad(embedding, ((0, P_PAD - P), (0, 0)))
    ps3, xnr3, yn, mbs = _k1(emb_pad, memory_bank)
    ps = ps3.reshape(P_PAD)[:P]
    xn_row = xnr3.reshape(1, P_PAD)

    ps4 = ps.reshape(B, NP_B)
    psimg = ps.reshape(B * HP, WP)
    l_mat = jnp.asarray(_L)
    lt_mat = l_mat.T
    amap, pred = _k2(ps4, psimg, emb_pad, mbs, xn_row, yn, l_mat, lt_mat)
    return amap.reshape(B, 1, OUT, OUT), pred.reshape(B)
